# baseline (speedup 1.0000x reference)
import functools

import jax
import jax.numpy as jnp
from jax import lax
from jax.experimental import pallas as pl
from jax.experimental.pallas import tpu as pltpu
from jax.experimental.pallas import tpu_sc as plsc

M = 8192
D = 8192
B = 1024
L = 16
NC = 2
NS = 16
NW = NC * NS
RPW = B // NW
GROUP = 4
NGROUP = RPW // GROUP

_mesh = plsc.VectorSubcoreMesh(core_axis_name="c", subcore_axis_name="s")


@functools.partial(
    pl.kernel,
    mesh=_mesh,
    out_type=jax.ShapeDtypeStruct((B, D), jnp.float32),
    scratch_types=[
        pltpu.VMEM((RPW,), jnp.int32),
        pltpu.SemaphoreType.DMA,
        pltpu.SemaphoreType.DMA,
    ],
    compiler_params=pltpu.CompilerParams(needs_layout_passes=False),
)
def _h2h_sc(mem_hbm, wval_hbm, widx_hbm, ridx_hbm, out_hbm,
            ridx_v, sem0, sem1):
    wid = lax.axis_index("s") * NC + lax.axis_index("c")
    base = wid * RPW

    pltpu.sync_copy(ridx_hbm.at[pl.ds(base, RPW)], ridx_v)

    iota = lax.iota(jnp.int32, L)
    rvec0 = ridx_v[pl.ds(0, L)]
    rvec1 = ridx_v[pl.ds(L, L)]
    NEG = jnp.int32(-(2 ** 31))
    sems = (sem0, sem1)

    def fire(g):
        descs = []
        for r in range(GROUP):
            i = g * GROUP + r
            vr = rvec0 if i < L else rvec1
            sr = jnp.max(jnp.where(iota == i % L, vr, NEG))
            descs.append(pltpu.async_copy(
                mem_hbm.at[pl.ds(sr, 1)],
                out_hbm.at[pl.ds(base + i, 1)], sems[g % 2]))
        return descs

    prev = fire(0)
    for g in range(NGROUP):
        nxt = fire(g + 1) if g + 1 < NGROUP else None
        for d in prev:
            d.wait()
        prev = nxt


def kernel(memory, write_val, write_idx, read_idx):
    return _h2h_sc(memory, write_val, write_idx, read_idx)


# speedup vs baseline: 20.2630x; 20.2630x over previous
import functools

import jax
import jax.numpy as jnp
from jax import lax
from jax.experimental import pallas as pl
from jax.experimental.pallas import tpu as pltpu
from jax.experimental.pallas import tpu_sc as plsc

M = 8192
D = 8192
B = 1024
L = 16
NC = 2
NS = 16
NW = NC * NS
RPW = B // NW
GROUP = 4
NGROUP = RPW // GROUP
NBUF = 3

_mesh = plsc.VectorSubcoreMesh(core_axis_name="c", subcore_axis_name="s")


def _dyn_gather(x, idx):
    dnums = lax.GatherDimensionNumbers(
        offset_dims=(), collapsed_slice_dims=(0,), start_index_map=(0,))
    return lax.gather(x, idx[:, None], dnums, slice_sizes=(1,),
                      mode=lax.GatherScatterMode.PROMISE_IN_BOUNDS)


@functools.partial(
    pl.kernel,
    mesh=_mesh,
    out_type=jax.ShapeDtypeStruct((B, D), jnp.float32),
    scratch_types=[
        pltpu.VMEM((RPW,), jnp.int32),
        pltpu.VMEM((B,), jnp.int32),
        pltpu.VMEM((M,), jnp.int32),
        pltpu.VMEM((GROUP, D), jnp.float32),
        pltpu.VMEM((GROUP, D), jnp.float32),
        pltpu.VMEM((GROUP, D), jnp.float32),
        pltpu.SemaphoreType.DMA,
        pltpu.SemaphoreType.DMA,
        pltpu.SemaphoreType.DMA,
        pltpu.SemaphoreType.DMA,
        pltpu.SemaphoreType.DMA,
        pltpu.SemaphoreType.DMA,
        pltpu.SemaphoreType.DMA,
    ],
    compiler_params=pltpu.CompilerParams(needs_layout_passes=False),
)
def _blind_memory_sc(mem_hbm, wval_hbm, widx_hbm, ridx_hbm, out_hbm,
                     ridx_v, widx_v, slot_v, buf0, buf1, buf2,
                     ldsem0, ldsem1, ldsem2, wbsem0, wbsem1, wbsem2, ovsem):
    wid = lax.axis_index("s") * NC + lax.axis_index("c")
    base = wid * RPW

    iota = lax.iota(jnp.int32, L)
    NEG = jnp.int32(-(2 ** 31))

    def lane_scalar(vec, lane):
        return jnp.max(jnp.where(iota == lane, vec, NEG))

    bufs = (buf0, buf1, buf2)
    ldsems = (ldsem0, ldsem1, ldsem2)
    wbsems = (wbsem0, wbsem1, wbsem2)

    pltpu.sync_copy(ridx_hbm.at[pl.ds(base, RPW)], ridx_v)
    rvec0 = ridx_v[pl.ds(0, L)]
    rvec1 = ridx_v[pl.ds(L, L)]

    def fire_mem_loads(g):
        buf, sem = bufs[g % NBUF], ldsems[g % NBUF]
        for r in range(GROUP):
            i = g * GROUP + r
            vr = rvec0 if i < L else rvec1
            sr = lane_scalar(vr, i % L)
            pltpu.async_copy(mem_hbm.at[pl.ds(sr, 1)],
                             buf.at[pl.ds(r, 1)], sem)

    for g in range(NBUF - 1):
        fire_mem_loads(g)

    pltpu.sync_copy(widx_hbm, widx_v)

    neg1 = jnp.full((L,), -1, jnp.int32)

    def init_body(i, carry):
        for u in range(4):
            slot_v[pl.ds(i * (4 * L) + u * L, L)] = neg1
        return carry

    lax.fori_loop(0, M // (4 * L), init_body, 0)

    def scat_body(w, carry):
        wvec = widx_v[pl.ds(w * L, L)]
        jv = iota + w * L
        maxj = jv
        for s in (1, 2, 4, 8):
            ridx = jnp.bitwise_and(iota + s, L - 1)
            rot_w = _dyn_gather(wvec, ridx)
            rot_m = _dyn_gather(maxj, ridx)
            maxj = jnp.where(rot_w == wvec, jnp.maximum(maxj, rot_m), maxj)
        keep = jv == maxj
        plsc.store_scatter(slot_v, [wvec], jv, mask=keep)
        return carry

    lax.fori_loop(0, B // L, scat_body, 0)

    jst0 = plsc.load_gather(slot_v, [rvec0])
    jst1 = plsc.load_gather(slot_v, [rvec1])

    def fire_loads(g):
        buf, sem = bufs[g % NBUF], ldsems[g % NBUF]
        for r in range(GROUP):
            i = g * GROUP + r
            vj = jst0 if i < L else jst1
            vr = rvec0 if i < L else rvec1
            lane = i % L
            sj = lane_scalar(vj, lane)
            sr = lane_scalar(vr, lane)

            @pl.when(sj >= 0)
            def _():
                pltpu.async_copy(wval_hbm.at[pl.ds(sj, 1)],
                                 buf.at[pl.ds(r, 1)], sem)

            @pl.when(sj < 0)
            def _():
                pltpu.async_copy(mem_hbm.at[pl.ds(sr, 1)],
                                 buf.at[pl.ds(r, 1)], sem)

    for g in range(NGROUP):
        nbuf = g % NBUF
        buf = bufs[nbuf]
        if g + NBUF - 1 < NGROUP:
            nxt = (g + NBUF - 1) % NBUF
            if g >= 1:
                pltpu.make_async_copy(out_hbm.at[pl.ds(0, GROUP)],
                                      bufs[nxt], wbsems[nxt]).wait()
            fire_loads(g + NBUF - 1)
        pltpu.make_async_copy(mem_hbm.at[pl.ds(0, GROUP)], buf,
                              ldsems[nbuf]).wait()
        if g < NBUF - 1:
            for r in range(GROUP):
                i = g * GROUP + r
                vj = jst0 if i < L else jst1
                sj = lane_scalar(vj, i % L)

                @pl.when(sj >= 0)
                def _():
                    pltpu.async_copy(wval_hbm.at[pl.ds(sj, 1)],
                                     buf.at[pl.ds(r, 1)], ovsem).wait()
        pltpu.async_copy(buf, out_hbm.at[pl.ds(base + g * GROUP, GROUP)],
                         wbsems[nbuf])

    for k in range(min(NBUF, NGROUP)):
        nbuf = (NGROUP - 1 - k) % NBUF
        pltpu.make_async_copy(out_hbm.at[pl.ds(0, GROUP)], bufs[nbuf],
                              wbsems[nbuf]).wait()


def kernel(memory, write_val, write_idx, read_idx):
    return _blind_memory_sc(memory, write_val, write_idx, read_idx)


# speedup vs baseline: 21.5235x; 1.0622x over previous
import functools

import jax
import jax.numpy as jnp
from jax import lax
from jax.experimental import pallas as pl
from jax.experimental.pallas import tpu as pltpu
from jax.experimental.pallas import tpu_sc as plsc

M = 8192
D = 8192
B = 1024
L = 16
NC = 2
NS = 16
NW = NC * NS
RPW = B // NW
GROUP = 4
NGROUP = RPW // GROUP
NBUF = 3

_mesh = plsc.VectorSubcoreMesh(core_axis_name="c", subcore_axis_name="s")


def _dyn_gather(x, idx):
    dnums = lax.GatherDimensionNumbers(
        offset_dims=(), collapsed_slice_dims=(0,), start_index_map=(0,))
    return lax.gather(x, idx[:, None], dnums, slice_sizes=(1,),
                      mode=lax.GatherScatterMode.PROMISE_IN_BOUNDS)


@functools.partial(
    pl.kernel,
    mesh=_mesh,
    out_type=jax.ShapeDtypeStruct((B, D), jnp.float32),
    scratch_types=[
        pltpu.VMEM((RPW,), jnp.int32),
        pltpu.VMEM((B,), jnp.int32),
        pltpu.VMEM((M,), jnp.int32),
        pltpu.VMEM((GROUP, D), jnp.float32),
        pltpu.VMEM((GROUP, D), jnp.float32),
        pltpu.VMEM((GROUP, D), jnp.float32),
        pltpu.SemaphoreType.DMA,
        pltpu.SemaphoreType.DMA,
        pltpu.SemaphoreType.DMA,
        pltpu.SemaphoreType.DMA,
        pltpu.SemaphoreType.DMA,
        pltpu.SemaphoreType.DMA,
    ],
    compiler_params=pltpu.CompilerParams(needs_layout_passes=False),
)
def _blind_memory_sc(mem_hbm, wval_hbm, widx_hbm, ridx_hbm, out_hbm,
                     ridx_v, widx_v, slot_v, buf0, buf1, buf2,
                     ldsem0, ldsem1, ldsem2, wbsem0, wbsem1, wbsem2):
    wid = lax.axis_index("s") * NC + lax.axis_index("c")
    base = wid * RPW

    pltpu.sync_copy(ridx_hbm.at[pl.ds(base, RPW)], ridx_v)
    pltpu.sync_copy(widx_hbm, widx_v)

    iota = lax.iota(jnp.int32, L)
    neg1 = jnp.full((L,), -1, jnp.int32)

    def init_body(i, carry):
        for u in range(4):
            slot_v[pl.ds(i * (4 * L) + u * L, L)] = neg1
        return carry

    lax.fori_loop(0, M // (4 * L), init_body, 0)

    def scat_body(w, carry):
        wvec = widx_v[pl.ds(w * L, L)]
        jv = iota + w * L
        maxj = jv
        for s in (1, 2, 4, 8):
            ridx = jnp.bitwise_and(iota + s, L - 1)
            rot_w = _dyn_gather(wvec, ridx)
            rot_m = _dyn_gather(maxj, ridx)
            maxj = jnp.where(rot_w == wvec, jnp.maximum(maxj, rot_m), maxj)
        keep = jv == maxj
        plsc.store_scatter(slot_v, [wvec], jv, mask=keep)
        return carry

    lax.fori_loop(0, B // L, scat_body, 0)

    rvec0 = ridx_v[pl.ds(0, L)]
    rvec1 = ridx_v[pl.ds(L, L)]
    jst0 = plsc.load_gather(slot_v, [rvec0])
    jst1 = plsc.load_gather(slot_v, [rvec1])

    NEG = jnp.int32(-(2 ** 31))

    def lane_scalar(vec, lane):
        return jnp.max(jnp.where(iota == lane, vec, NEG))

    bufs = (buf0, buf1, buf2)
    ldsems = (ldsem0, ldsem1, ldsem2)
    wbsems = (wbsem0, wbsem1, wbsem2)

    def fire_loads(g):
        buf, sem = bufs[g % NBUF], ldsems[g % NBUF]
        for r in range(GROUP):
            i = g * GROUP + r
            vj = jst0 if i < L else jst1
            vr = rvec0 if i < L else rvec1
            lane = i % L
            sj = lane_scalar(vj, lane)
            sr = lane_scalar(vr, lane)

            @pl.when(sj >= 0)
            def _():
                pltpu.async_copy(wval_hbm.at[pl.ds(sj, 1)],
                                 buf.at[pl.ds(r, 1)], sem)

            @pl.when(sj < 0)
            def _():
                pltpu.async_copy(mem_hbm.at[pl.ds(sr, 1)],
                                 buf.at[pl.ds(r, 1)], sem)

    for g in range(NBUF - 1):
        fire_loads(g)
    for g in range(NGROUP):
        nbuf = g % NBUF
        buf = bufs[nbuf]
        if g + NBUF - 1 < NGROUP:
            nxt = (g + NBUF - 1) % NBUF
            if g >= 1:
                pltpu.make_async_copy(out_hbm.at[pl.ds(0, GROUP)],
                                      bufs[nxt], wbsems[nxt]).wait()
            fire_loads(g + NBUF - 1)
        pltpu.make_async_copy(mem_hbm.at[pl.ds(0, GROUP)], buf,
                              ldsems[nbuf]).wait()
        pltpu.async_copy(buf, out_hbm.at[pl.ds(base + g * GROUP, GROUP)],
                         wbsems[nbuf])

    for k in range(min(NBUF, NGROUP)):
        nbuf = (NGROUP - 1 - k) % NBUF
        pltpu.make_async_copy(out_hbm.at[pl.ds(0, GROUP)], bufs[nbuf],
                              wbsems[nbuf]).wait()


def kernel(memory, write_val, write_idx, read_idx):
    return _blind_memory_sc(memory, write_val, write_idx, read_idx)
